# baseline (speedup 1.0000x reference)
import jax
import jax.numpy as jnp
from jax.experimental import pallas as pl
from jax.experimental.pallas import tpu as pltpu

_GRID = 1


def _trim_kernel(x_ref, v_ref, m_ref, xo_ref, vo_ref, mo_ref):
    xo_ref[...] = x_ref[...]

    @pl.when(pl.program_id(0) == 0)
    def _():
        vo_ref[...] = v_ref[...]
        mo_ref[...] = m_ref[...] != 0.0


def kernel(x, v, mask):
    b, n, l = x.shape
    _, nv, _ = v.shape
    _, nm, _ = mask.shape
    rows = b * n
    blk = rows // _GRID
    x2 = x.reshape(rows, l)
    xo, vo, mo = pl.pallas_call(
        _trim_kernel,
        grid=(_GRID,),
        in_specs=[
            pl.BlockSpec((blk, l), lambda i: (i, 0)),
            pl.BlockSpec((b, nv, l), lambda i: (0, 0, 0)),
            pl.BlockSpec((b, nm, l), lambda i: (0, 0, 0)),
        ],
        out_specs=[
            pl.BlockSpec((blk, l), lambda i: (i, 0)),
            pl.BlockSpec((b, nv, l), lambda i: (0, 0, 0)),
            pl.BlockSpec((b, nm, l), lambda i: (0, 0, 0)),
        ],
        out_shape=[
            jax.ShapeDtypeStruct((rows, l), x.dtype),
            jax.ShapeDtypeStruct(v.shape, v.dtype),
            jax.ShapeDtypeStruct(mask.shape, jnp.bool_),
        ],
    )(x2, v, mask)
    return (xo.reshape(x.shape), vo, mo)
